# baseline (speedup 1.0000x reference)
import functools

import jax
import jax.numpy as jnp
from jax import lax
from jax.experimental import pallas as pl
from jax.experimental.pallas import tpu as pltpu
from jax.experimental.pallas import tpu_sc as plsc

N_NODES = 1000000
N_PHYS = 800000
NBX = 512
NBY = 512
STRETCH = 1.4142135623730951
OUT_SCALE = 0.5

N_TILES = 32
SHARE = 25600
N_PAD = N_TILES * SHARE
CHUNK = 1280
N_CHUNKS = SHARE // CHUNK
BATCHES = CHUNK // 16
QROWS = NBX // 4
NREC = 7
DEPTH = 512
SDEPTH = DEPTH + 16
MAXBLK = SHARE // DEPTH + 2
MBLK = 16384


def _tc_records(x, y, nsx, nsy, pw):
    def body(x_ref, y_ref, sx_ref, sy_ref, pw_ref, o_ref):
        x = x_ref[...]
        y = y_ref[...]
        nsx = sx_ref[...]
        nsy = sy_ref[...]
        pw = pw_ref[...]
        hx = 0.5 * jnp.maximum(nsx, STRETCH)
        hy = 0.5 * jnp.maximum(nsy, STRETCH)
        xc = x + 0.5 * nsx
        yc = y + 0.5 * nsy
        x_min = xc - hx
        x_max = xc + hx
        y_min = yc - hy
        y_max = yc + hy
        bxl = jnp.maximum(x_min.astype(jnp.int32), 0)
        byl = jnp.maximum(y_min.astype(jnp.int32), 0)
        bxf = bxl.astype(jnp.float32)
        byf = byl.astype(jnp.float32)
        dens = pw / ((4.0 * hx) * hy)
        tx0 = bxf + 1.0
        tx1 = bxf + 2.0
        ty0 = byf + 1.0
        ty1 = byf + 2.0
        o_ref[0, :] = tx0 - jnp.maximum(x_min, bxf)
        o_ref[1, :] = jnp.minimum(x_max, tx1) - tx0
        o_ref[2, :] = jnp.maximum(x_max - tx1, 0.0)
        o_ref[3, :] = (ty0 - jnp.maximum(y_min, byf)) * dens
        o_ref[4, :] = (jnp.minimum(y_max, ty1) - ty0) * dens
        o_ref[5, :] = jnp.maximum(y_max - ty1, 0.0) * dens
        o_ref[6, :] = lax.bitcast_convert_type(bxl * 1024 + byl, jnp.float32)

    spec = pl.BlockSpec((MBLK,), lambda i: (i,))
    return pl.pallas_call(
        body,
        grid=(N_PAD // MBLK,),
        in_specs=[spec] * 5,
        out_specs=pl.BlockSpec((NREC, MBLK), lambda i: (0, i)),
        out_shape=jax.ShapeDtypeStruct((NREC, N_PAD), jnp.float32),
    )(x, y, nsx, nsy, pw)


def _sc_partials():
    mesh = plsc.VectorSubcoreMesh(core_axis_name="c", subcore_axis_name="s")

    @functools.partial(
        pl.kernel,
        mesh=mesh,
        compiler_params=pltpu.CompilerParams(needs_layout_passes=False),
        out_type=(
            jax.ShapeDtypeStruct((N_TILES, NBX, NBY), jnp.float32),
            jax.ShapeDtypeStruct((N_TILES, 4, MAXBLK, NREC * SDEPTH),
                                 jnp.float32),
        ),
        scratch_types=[
            pltpu.VMEM((QROWS, NBY), jnp.float32),
            pltpu.VMEM((NREC, CHUNK), jnp.float32),
            pltpu.VMEM((NREC, CHUNK), jnp.float32),
            pltpu.VMEM((NREC * SDEPTH,), jnp.float32),
            pltpu.VMEM((NREC * SDEPTH,), jnp.float32),
            pltpu.VMEM((NREC * SDEPTH,), jnp.float32),
            pltpu.VMEM((NREC * SDEPTH,), jnp.float32),
            pltpu.VMEM((NREC * SDEPTH,), jnp.float32),
            pltpu.SemaphoreType.DMA,
            pltpu.SemaphoreType.DMA,
        ],
    )
    def body(rec_hbm, out_hbm, bkt_hbm, acc, gbuf0, gbuf1,
             stg0, stg1, stg2, stg3, abuf, sem0, sem1):
        stg = (stg0, stg1, stg2, stg3)
        wid = lax.axis_index("s") * 2 + lax.axis_index("c")
        base = wid * SHARE
        zeros16 = jnp.zeros((16,), jnp.float32)
        lane = lax.broadcasted_iota(jnp.int32, (16,), 0)

        def start(c, buf, sem):
            off = base + c * CHUNK
            pltpu.async_copy(rec_hbm.at[:, pl.ds(off, CHUNK)], buf, sem)

        def wait(c, buf, sem):
            off = base + c * CHUNK
            pltpu.make_async_copy(
                rec_hbm.at[:, pl.ds(off, CHUNK)], buf, sem).wait()

        def gen_chunk(buf, carry):
            def gen2(bi, carry):
                for u in range(2):
                    s = pl.ds((bi * 2 + u) * 16, 16)
                    f = [buf[a, s] for a in range(NREC)]
                    bb = plsc.bitcast(f[6], jnp.int32)
                    bxl = bb >> 10
                    qa = bxl >> 7
                    qb = jnp.minimum(bxl + 2, NBX - 1) >> 7
                    new = []
                    for q in range(4):
                        cur = carry[q]
                        nb = carry[4 + q]
                        mq = (qa == q) | (qb == q)
                        pfx = plsc.cumsum(mq.astype(jnp.int32))
                        idx = (cur - 1) + pfx
                        for a in range(NREC):
                            plsc.store_scatter(
                                stg[q], [idx + a * SDEPTH], f[a], mask=mq)
                        ncur = cur + jnp.sum(mq.astype(jnp.int32))
                        do_flush = ncur >= DEPTH

                        @pl.when(do_flush)
                        def _(nb=nb, q=q):
                            pltpu.sync_copy(stg[q],
                                            bkt_hbm.at[wid, q, nb])
                            for a in range(NREC):
                                stg[q][pl.ds(a * SDEPTH, 16)] = (
                                    stg[q][pl.ds(a * SDEPTH + DEPTH, 16)])

                        ncur2 = jnp.where(do_flush, ncur - DEPTH, ncur)
                        nnb = jnp.where(do_flush, nb + 1, nb)
                        new += [ncur2, nnb]
                    carry = (new[0], new[2], new[4], new[6],
                             new[1], new[3], new[5], new[7])
                return carry

            return lax.fori_loop(0, BATCHES // 2, gen2, carry)

        def gen_ring(c2, carry):
            c = c2 * 2
            wait(c, gbuf0, sem0)
            start(c + 1, gbuf1, sem1)
            carry = gen_chunk(gbuf0, carry)
            wait(c + 1, gbuf1, sem1)

            @pl.when(c2 < N_CHUNKS // 2 - 1)
            def _():
                start(c + 2, gbuf0, sem0)

            return gen_chunk(gbuf1, carry)

        start(0, gbuf0, sem0)
        z = jnp.int32(0)
        carry = lax.fori_loop(0, N_CHUNKS // 2, gen_ring,
                              (z, z, z, z, z, z, z, z))
        totals = []
        for q in range(4):
            pltpu.sync_copy(stg[q], bkt_hbm.at[wid, q, carry[4 + q]])
            totals.append(carry[4 + q] * DEPTH + carry[q])

        for q in range(4):
            qlo = q * QROWS
            tot = totals[q]

            def zero_row(r, _):
                for cgrp in range(NBY // 16):
                    acc[r, pl.ds(cgrp * 16, 16)] = zeros16
                return 0

            lax.fori_loop(0, QROWS, zero_row, 0)

            def blk_body(blk, _, q=q, qlo=qlo, tot=tot):
                pltpu.sync_copy(bkt_hbm.at[wid, q, blk], abuf)
                gbase = blk * DEPTH

                def ab2(bi, _):
                    for u in range(2):
                        bat = bi * 2 + u
                        o = bat * 16

                        def fld(a, o=o):
                            return abuf[pl.ds(a * SDEPTH + o, 16)]

                        ovx = (fld(0), fld(1), fld(2))
                        wy = (fld(3), fld(4), fld(5))
                        bb = plsc.bitcast(fld(6), jnp.int32)
                        bxl = bb >> 10
                        byl = bb & 1023
                        vmask = (gbase + bat * 16) + lane < tot
                        iy = (byl, byl + 1, jnp.minimum(byl + 2, NBY - 1))
                        ix = (bxl, bxl + 1, jnp.minimum(bxl + 2, NBX - 1))
                        for j in range(3):
                            ixl = ix[j] - qlo
                            msk = (plsc.bitcast(ixl, jnp.uint32)
                                   < jnp.uint32(QROWS)) & vmask
                            for k in range(3):
                                plsc.addupdate_scatter(
                                    acc, [ixl, iy[k]], ovx[j] * wy[k],
                                    mask=msk)
                    return 0

                lax.fori_loop(0, DEPTH // 32, ab2, 0)
                return 0

            trip = (tot + DEPTH - 1) >> 9
            lax.fori_loop(0, trip, blk_body, 0)
            pltpu.sync_copy(acc, out_hbm.at[wid, pl.ds(qlo, QROWS)])

    return body


def _tc_reduce(partials):
    def body(p_ref, o_ref):
        o_ref[...] = jnp.sum(p_ref[...], axis=0) * OUT_SCALE

    return pl.pallas_call(
        body,
        grid=(NBX // 8,),
        in_specs=[pl.BlockSpec((N_TILES, 8, NBY), lambda i: (0, i, 0))],
        out_specs=pl.BlockSpec((8, NBY), lambda i: (i, 0)),
        out_shape=jax.ShapeDtypeStruct((NBX, NBY), jnp.float32),
    )(partials)


@jax.jit
def kernel(pos, node_size_x, node_size_y, pin_weights):
    x = pos[:N_PHYS]
    y = pos[N_NODES:N_NODES + N_PHYS]
    nsx = node_size_x[:N_PHYS]
    nsy = node_size_y[:N_PHYS]
    pad = N_PAD - N_PHYS
    half = jnp.full((pad,), 0.5, jnp.float32)
    zero = jnp.zeros((pad,), jnp.float32)
    records = _tc_records(
        jnp.concatenate([x, half]),
        jnp.concatenate([y, half]),
        jnp.concatenate([nsx, half]),
        jnp.concatenate([nsy, half]),
        jnp.concatenate([pin_weights, zero]),
    )
    partials, _ = _sc_partials()(records)
    return _tc_reduce(partials)


# speedup vs baseline: 1.0003x; 1.0003x over previous
import functools

import jax
import jax.numpy as jnp
from jax import lax
from jax.experimental import pallas as pl
from jax.experimental.pallas import tpu as pltpu
from jax.experimental.pallas import tpu_sc as plsc

N_NODES = 1000000
N_PHYS = 800000
NBX = 512
NBY = 512
STRETCH = 1.4142135623730951
OUT_SCALE = 0.5

N_TILES = 32
SHARE = 25600
N_PAD = N_TILES * SHARE
CHUNK = 1280
N_CHUNKS = SHARE // CHUNK
BATCHES = CHUNK // 16
QROWS = NBX // 4
NREC = 7
DEPTH = 512
SDEPTH = DEPTH + 16
MAXBLK = SHARE // DEPTH + 2
MBLK = 16384


def _tc_records(x, y, nsx, nsy, pw):
    def body(x_ref, y_ref, sx_ref, sy_ref, pw_ref, o_ref):
        x = x_ref[...]
        y = y_ref[...]
        nsx = sx_ref[...]
        nsy = sy_ref[...]
        pw = pw_ref[...]
        hx = 0.5 * jnp.maximum(nsx, STRETCH)
        hy = 0.5 * jnp.maximum(nsy, STRETCH)
        xc = x + 0.5 * nsx
        yc = y + 0.5 * nsy
        x_min = xc - hx
        x_max = xc + hx
        y_min = yc - hy
        y_max = yc + hy
        bxl = jnp.maximum(x_min.astype(jnp.int32), 0)
        byl = jnp.maximum(y_min.astype(jnp.int32), 0)
        bxf = bxl.astype(jnp.float32)
        byf = byl.astype(jnp.float32)
        dens = pw / ((4.0 * hx) * hy)
        tx0 = bxf + 1.0
        tx1 = bxf + 2.0
        ty0 = byf + 1.0
        ty1 = byf + 2.0
        o_ref[0, :] = tx0 - jnp.maximum(x_min, bxf)
        o_ref[1, :] = jnp.minimum(x_max, tx1) - tx0
        o_ref[2, :] = jnp.maximum(x_max - tx1, 0.0)
        o_ref[3, :] = (ty0 - jnp.maximum(y_min, byf)) * dens
        o_ref[4, :] = (jnp.minimum(y_max, ty1) - ty0) * dens
        o_ref[5, :] = jnp.maximum(y_max - ty1, 0.0) * dens
        o_ref[6, :] = lax.bitcast_convert_type(bxl * 1024 + byl, jnp.float32)

    spec = pl.BlockSpec((MBLK,), lambda i: (i,))
    return pl.pallas_call(
        body,
        grid=(N_PAD // MBLK,),
        in_specs=[spec] * 5,
        out_specs=pl.BlockSpec((NREC, MBLK), lambda i: (0, i)),
        out_shape=jax.ShapeDtypeStruct((NREC, N_PAD), jnp.float32),
    )(x, y, nsx, nsy, pw)


def _sc_partials():
    mesh = plsc.VectorSubcoreMesh(core_axis_name="c", subcore_axis_name="s")

    @functools.partial(
        pl.kernel,
        mesh=mesh,
        compiler_params=pltpu.CompilerParams(needs_layout_passes=False),
        out_type=(
            jax.ShapeDtypeStruct((N_TILES, NBX, NBY), jnp.float32),
            jax.ShapeDtypeStruct((N_TILES, 4, MAXBLK, NREC * SDEPTH),
                                 jnp.float32),
        ),
        scratch_types=[
            pltpu.VMEM((QROWS, NBY), jnp.float32),
            pltpu.VMEM((NREC, CHUNK), jnp.float32),
            pltpu.VMEM((NREC, CHUNK), jnp.float32),
            pltpu.VMEM((NREC * SDEPTH,), jnp.float32),
            pltpu.VMEM((NREC * SDEPTH,), jnp.float32),
            pltpu.VMEM((NREC * SDEPTH,), jnp.float32),
            pltpu.VMEM((NREC * SDEPTH,), jnp.float32),
            pltpu.VMEM((NREC * SDEPTH,), jnp.float32),
            pltpu.SemaphoreType.DMA,
            pltpu.SemaphoreType.DMA,
        ],
    )
    def body(rec_hbm, out_hbm, bkt_hbm, acc, gbuf0, gbuf1,
             stg0, stg1, stg2, stg3, abuf, sem0, sem1):
        stg = (stg0, stg1, stg2, stg3)
        wid = lax.axis_index("s") * 2 + lax.axis_index("c")
        base = wid * SHARE
        zeros16 = jnp.zeros((16,), jnp.float32)
        lane = lax.broadcasted_iota(jnp.int32, (16,), 0)

        def start(c, buf, sem):
            off = base + c * CHUNK
            pltpu.async_copy(rec_hbm.at[:, pl.ds(off, CHUNK)], buf, sem)

        def wait(c, buf, sem):
            off = base + c * CHUNK
            pltpu.make_async_copy(
                rec_hbm.at[:, pl.ds(off, CHUNK)], buf, sem).wait()

        def gen_chunk(buf, carry):
            def gen2(bi, carry):
                for u in range(2):
                    s = pl.ds((bi * 2 + u) * 16, 16)
                    f = [buf[a, s] for a in range(NREC)]
                    bb = plsc.bitcast(f[6], jnp.int32)
                    bxl = bb >> 10
                    qa = bxl >> 7
                    qb = jnp.minimum(bxl + 2, NBX - 1) >> 7
                    new = []
                    for q in range(4):
                        cur = carry[q]
                        nb = carry[4 + q]
                        mq = (qa == q) | (qb == q)
                        for a in range(NREC):
                            plsc.store_compressed(
                                stg[q].at[pl.ds(cur + a * SDEPTH, 16)],
                                f[a], mask=mq)
                        ncur = cur + jnp.sum(mq.astype(jnp.int32))
                        do_flush = ncur >= DEPTH

                        @pl.when(do_flush)
                        def _(nb=nb, q=q):
                            pltpu.sync_copy(stg[q],
                                            bkt_hbm.at[wid, q, nb])
                            for a in range(NREC):
                                stg[q][pl.ds(a * SDEPTH, 16)] = (
                                    stg[q][pl.ds(a * SDEPTH + DEPTH, 16)])

                        ncur2 = jnp.where(do_flush, ncur - DEPTH, ncur)
                        nnb = jnp.where(do_flush, nb + 1, nb)
                        new += [ncur2, nnb]
                    carry = (new[0], new[2], new[4], new[6],
                             new[1], new[3], new[5], new[7])
                return carry

            return lax.fori_loop(0, BATCHES // 2, gen2, carry)

        def gen_ring(c2, carry):
            c = c2 * 2
            wait(c, gbuf0, sem0)
            start(c + 1, gbuf1, sem1)
            carry = gen_chunk(gbuf0, carry)
            wait(c + 1, gbuf1, sem1)

            @pl.when(c2 < N_CHUNKS // 2 - 1)
            def _():
                start(c + 2, gbuf0, sem0)

            return gen_chunk(gbuf1, carry)

        start(0, gbuf0, sem0)
        z = jnp.int32(0)
        carry = lax.fori_loop(0, N_CHUNKS // 2, gen_ring,
                              (z, z, z, z, z, z, z, z))
        totals = []
        for q in range(4):
            pltpu.sync_copy(stg[q], bkt_hbm.at[wid, q, carry[4 + q]])
            totals.append(carry[4 + q] * DEPTH + carry[q])

        for q in range(4):
            qlo = q * QROWS
            tot = totals[q]

            def zero_row(r, _):
                for cgrp in range(NBY // 16):
                    acc[r, pl.ds(cgrp * 16, 16)] = zeros16
                return 0

            lax.fori_loop(0, QROWS, zero_row, 0)

            def blk_body(blk, _, q=q, qlo=qlo, tot=tot):
                pltpu.sync_copy(bkt_hbm.at[wid, q, blk], abuf)
                gbase = blk * DEPTH

                def ab2(bi, _):
                    for u in range(2):
                        bat = bi * 2 + u
                        o = bat * 16

                        def fld(a, o=o):
                            return abuf[pl.ds(a * SDEPTH + o, 16)]

                        ovx = (fld(0), fld(1), fld(2))
                        wy = (fld(3), fld(4), fld(5))
                        bb = plsc.bitcast(fld(6), jnp.int32)
                        bxl = bb >> 10
                        byl = bb & 1023
                        vmask = (gbase + bat * 16) + lane < tot
                        iy = (byl, byl + 1, jnp.minimum(byl + 2, NBY - 1))
                        ix = (bxl, bxl + 1, jnp.minimum(bxl + 2, NBX - 1))
                        for j in range(3):
                            ixl = ix[j] - qlo
                            msk = (plsc.bitcast(ixl, jnp.uint32)
                                   < jnp.uint32(QROWS)) & vmask
                            for k in range(3):
                                plsc.addupdate_scatter(
                                    acc, [ixl, iy[k]], ovx[j] * wy[k],
                                    mask=msk)
                    return 0

                lax.fori_loop(0, DEPTH // 32, ab2, 0)
                return 0

            trip = (tot + DEPTH - 1) >> 9
            lax.fori_loop(0, trip, blk_body, 0)
            pltpu.sync_copy(acc, out_hbm.at[wid, pl.ds(qlo, QROWS)])

    return body


def _tc_reduce(partials):
    def body(p_ref, o_ref):
        o_ref[...] = jnp.sum(p_ref[...], axis=0) * OUT_SCALE

    return pl.pallas_call(
        body,
        grid=(NBX // 8,),
        in_specs=[pl.BlockSpec((N_TILES, 8, NBY), lambda i: (0, i, 0))],
        out_specs=pl.BlockSpec((8, NBY), lambda i: (i, 0)),
        out_shape=jax.ShapeDtypeStruct((NBX, NBY), jnp.float32),
    )(partials)


@jax.jit
def kernel(pos, node_size_x, node_size_y, pin_weights):
    x = pos[:N_PHYS]
    y = pos[N_NODES:N_NODES + N_PHYS]
    nsx = node_size_x[:N_PHYS]
    nsy = node_size_y[:N_PHYS]
    pad = N_PAD - N_PHYS
    half = jnp.full((pad,), 0.5, jnp.float32)
    zero = jnp.zeros((pad,), jnp.float32)
    records = _tc_records(
        jnp.concatenate([x, half]),
        jnp.concatenate([y, half]),
        jnp.concatenate([nsx, half]),
        jnp.concatenate([nsy, half]),
        jnp.concatenate([pin_weights, zero]),
    )
    partials, _ = _sc_partials()(records)
    return _tc_reduce(partials)


# speedup vs baseline: 1.2584x; 1.2580x over previous
import functools

import jax
import jax.numpy as jnp
from jax import lax
from jax.experimental import pallas as pl
from jax.experimental.pallas import tpu as pltpu
from jax.experimental.pallas import tpu_sc as plsc

N_NODES = 1000000
N_PHYS = 800000
NBX = 512
NBY = 512
STRETCH = 1.4142135623730951
OUT_SCALE = 0.5

N_TILES = 32
SHARE = 25600
N_PAD = N_TILES * SHARE
CHUNK = 3200
N_CHUNKS = SHARE // CHUNK
BATCHES = CHUNK // 16
QROWS = NBX // 4
NREC = 7
MBLK = 16384


def _tc_records(x, y, nsx, nsy, pw):
    def body(x_ref, y_ref, sx_ref, sy_ref, pw_ref, o_ref):
        x = x_ref[...]
        y = y_ref[...]
        nsx = sx_ref[...]
        nsy = sy_ref[...]
        pw = pw_ref[...]
        hx = 0.5 * jnp.maximum(nsx, STRETCH)
        hy = 0.5 * jnp.maximum(nsy, STRETCH)
        xc = x + 0.5 * nsx
        yc = y + 0.5 * nsy
        x_min = xc - hx
        x_max = xc + hx
        y_min = yc - hy
        y_max = yc + hy
        bxl = jnp.maximum(x_min.astype(jnp.int32), 0)
        byl = jnp.maximum(y_min.astype(jnp.int32), 0)
        bxf = bxl.astype(jnp.float32)
        byf = byl.astype(jnp.float32)
        dens = pw / ((4.0 * hx) * hy)
        tx0 = bxf + 1.0
        tx1 = bxf + 2.0
        ty0 = byf + 1.0
        ty1 = byf + 2.0
        o_ref[0, :] = tx0 - jnp.maximum(x_min, bxf)
        o_ref[1, :] = jnp.minimum(x_max, tx1) - tx0
        o_ref[2, :] = jnp.maximum(x_max - tx1, 0.0)
        o_ref[3, :] = (ty0 - jnp.maximum(y_min, byf)) * dens
        o_ref[4, :] = (jnp.minimum(y_max, ty1) - ty0) * dens
        o_ref[5, :] = jnp.maximum(y_max - ty1, 0.0) * dens
        o_ref[6, :] = lax.bitcast_convert_type(bxl * 1024 + byl, jnp.float32)

    spec = pl.BlockSpec((MBLK,), lambda i: (i,))
    return pl.pallas_call(
        body,
        grid=(N_PAD // MBLK,),
        in_specs=[spec] * 5,
        out_specs=pl.BlockSpec((NREC, MBLK), lambda i: (0, i)),
        out_shape=jax.ShapeDtypeStruct((NREC, N_PAD), jnp.float32),
    )(x, y, nsx, nsy, pw)


def _sc_partials():
    mesh = plsc.VectorSubcoreMesh(core_axis_name="c", subcore_axis_name="s")

    @functools.partial(
        pl.kernel,
        mesh=mesh,
        compiler_params=pltpu.CompilerParams(needs_layout_passes=False),
        out_type=jax.ShapeDtypeStruct((N_TILES, NBX, NBY), jnp.float32),
        scratch_types=[
            pltpu.VMEM((QROWS, NBY), jnp.float32),
            pltpu.VMEM((NREC, CHUNK), jnp.float32),
            pltpu.VMEM((NREC, CHUNK), jnp.float32),
            pltpu.SemaphoreType.DMA,
            pltpu.SemaphoreType.DMA,
        ],
    )
    def body(rec_hbm, out_hbm, acc, buf0, buf1, sem0, sem1):
        wid = lax.axis_index("s") * 2 + lax.axis_index("c")
        base = wid * SHARE
        zeros16 = jnp.zeros((16,), jnp.float32)

        def start(c, buf, sem):
            off = base + c * CHUNK
            pltpu.async_copy(rec_hbm.at[:, pl.ds(off, CHUNK)], buf, sem)

        def wait(c, buf, sem):
            off = base + c * CHUNK
            pltpu.make_async_copy(
                rec_hbm.at[:, pl.ds(off, CHUNK)], buf, sem).wait()

        def process(buf, qlo):
            def do2(bb_i, _):
                for u in range(4):
                    s = pl.ds((bb_i * 4 + u) * 16, 16)
                    ovx = (buf[0, s], buf[1, s], buf[2, s])
                    wy = (buf[3, s], buf[4, s], buf[5, s])
                    bb = plsc.bitcast(buf[6, s], jnp.int32)
                    bxl = bb >> 10
                    byl = bb & 1023
                    iy = (byl, byl + 1, jnp.minimum(byl + 2, NBY - 1))
                    ix = (bxl, bxl + 1, jnp.minimum(bxl + 2, NBX - 1))
                    for j in range(3):
                        ixl = ix[j] - qlo
                        msk = plsc.bitcast(ixl, jnp.uint32) < jnp.uint32(QROWS)
                        for k in range(3):
                            plsc.addupdate_scatter(
                                acc, [ixl, iy[k]], ovx[j] * wy[k], mask=msk)
                return 0

            lax.fori_loop(0, BATCHES // 4, do2, 0)

        def do_quadrant(q, _):
            qlo = q * QROWS
            start(0, buf0, sem0)

            def zero_row(r, _):
                for cgrp in range(NBY // 16):
                    acc[r, pl.ds(cgrp * 16, 16)] = zeros16
                return 0

            lax.fori_loop(0, QROWS, zero_row, 0)

            def ring(c2, _):
                c = c2 * 2
                wait(c, buf0, sem0)
                start(c + 1, buf1, sem1)
                process(buf0, qlo)
                wait(c + 1, buf1, sem1)

                @pl.when(c2 < N_CHUNKS // 2 - 1)
                def _():
                    start(c + 2, buf0, sem0)

                process(buf1, qlo)
                return 0

            lax.fori_loop(0, N_CHUNKS // 2, ring, 0)
            pltpu.sync_copy(acc, out_hbm.at[wid, pl.ds(qlo, QROWS)])
            return 0

        lax.fori_loop(0, 4, do_quadrant, 0)

    return body


def _tc_reduce(partials):
    def body(p_ref, o_ref):
        o_ref[...] = jnp.sum(p_ref[...], axis=0) * OUT_SCALE

    return pl.pallas_call(
        body,
        grid=(NBX // 8,),
        in_specs=[pl.BlockSpec((N_TILES, 8, NBY), lambda i: (0, i, 0))],
        out_specs=pl.BlockSpec((8, NBY), lambda i: (i, 0)),
        out_shape=jax.ShapeDtypeStruct((NBX, NBY), jnp.float32),
    )(partials)


@jax.jit
def kernel(pos, node_size_x, node_size_y, pin_weights):
    x = pos[:N_PHYS]
    y = pos[N_NODES:N_NODES + N_PHYS]
    nsx = node_size_x[:N_PHYS]
    nsy = node_size_y[:N_PHYS]
    pad = N_PAD - N_PHYS
    half = jnp.full((pad,), 0.5, jnp.float32)
    zero = jnp.zeros((pad,), jnp.float32)
    records = _tc_records(
        jnp.concatenate([x, half]),
        jnp.concatenate([y, half]),
        jnp.concatenate([nsx, half]),
        jnp.concatenate([nsy, half]),
        jnp.concatenate([pin_weights, zero]),
    )
    partials = _sc_partials()(records)
    return _tc_reduce(partials)


# speedup vs baseline: 1.3669x; 1.0862x over previous
import functools

import jax
import jax.numpy as jnp
from jax import lax
from jax.experimental import pallas as pl
from jax.experimental.pallas import tpu as pltpu
from jax.experimental.pallas import tpu_sc as plsc

N_NODES = 1000000
N_PHYS = 800000
NBX = 512
NBY = 512
STRETCH = 1.4142135623730951
OUT_SCALE = 0.5

N_TILES = 32
SHARE = 25600
N_PAD = N_TILES * SHARE
CHUNK = 1280
N_CHUNKS = SHARE // CHUNK
BATCHES = CHUNK // 16
QROWS = 176
NREC = 7
MBLK = 16384


def _tc_records(x, y, nsx, nsy, pw):
    def body(x_ref, y_ref, sx_ref, sy_ref, pw_ref, o_ref):
        x = x_ref[...]
        y = y_ref[...]
        nsx = sx_ref[...]
        nsy = sy_ref[...]
        pw = pw_ref[...]
        hx = 0.5 * jnp.maximum(nsx, STRETCH)
        hy = 0.5 * jnp.maximum(nsy, STRETCH)
        xc = x + 0.5 * nsx
        yc = y + 0.5 * nsy
        x_min = xc - hx
        x_max = xc + hx
        y_min = yc - hy
        y_max = yc + hy
        bxl = jnp.maximum(x_min.astype(jnp.int32), 0)
        byl = jnp.maximum(y_min.astype(jnp.int32), 0)
        bxf = bxl.astype(jnp.float32)
        byf = byl.astype(jnp.float32)
        dens = pw / ((4.0 * hx) * hy)
        tx0 = bxf + 1.0
        tx1 = bxf + 2.0
        ty0 = byf + 1.0
        ty1 = byf + 2.0
        o_ref[0, :] = tx0 - jnp.maximum(x_min, bxf)
        o_ref[1, :] = jnp.minimum(x_max, tx1) - tx0
        o_ref[2, :] = jnp.maximum(x_max - tx1, 0.0)
        o_ref[3, :] = (ty0 - jnp.maximum(y_min, byf)) * dens
        o_ref[4, :] = (jnp.minimum(y_max, ty1) - ty0) * dens
        o_ref[5, :] = jnp.maximum(y_max - ty1, 0.0) * dens
        o_ref[6, :] = lax.bitcast_convert_type(bxl * 1024 + byl, jnp.float32)

    spec = pl.BlockSpec((MBLK,), lambda i: (i,))
    return pl.pallas_call(
        body,
        grid=(N_PAD // MBLK,),
        in_specs=[spec] * 5,
        out_specs=pl.BlockSpec((NREC, MBLK), lambda i: (0, i)),
        out_shape=jax.ShapeDtypeStruct((NREC, N_PAD), jnp.float32),
    )(x, y, nsx, nsy, pw)


def _sc_partials():
    mesh = plsc.VectorSubcoreMesh(core_axis_name="c", subcore_axis_name="s")

    @functools.partial(
        pl.kernel,
        mesh=mesh,
        compiler_params=pltpu.CompilerParams(needs_layout_passes=False),
        out_type=jax.ShapeDtypeStruct((N_TILES, NBX, NBY), jnp.float32),
        scratch_types=[
            pltpu.VMEM((QROWS, NBY), jnp.float32),
            pltpu.VMEM((NREC, CHUNK), jnp.float32),
            pltpu.VMEM((NREC, CHUNK), jnp.float32),
            pltpu.SemaphoreType.DMA,
            pltpu.SemaphoreType.DMA,
        ],
    )
    def body(rec_hbm, out_hbm, acc, buf0, buf1, sem0, sem1):
        wid = lax.axis_index("s") * 2 + lax.axis_index("c")
        base = wid * SHARE
        zeros16 = jnp.zeros((16,), jnp.float32)

        def start(c, buf, sem):
            off = base + c * CHUNK
            pltpu.async_copy(rec_hbm.at[:, pl.ds(off, CHUNK)], buf, sem)

        def wait(c, buf, sem):
            off = base + c * CHUNK
            pltpu.make_async_copy(
                rec_hbm.at[:, pl.ds(off, CHUNK)], buf, sem).wait()

        def process(buf, qlo, qrows):
            def do2(bb_i, _):
                for u in range(4):
                    s = pl.ds((bb_i * 4 + u) * 16, 16)
                    ovx = (buf[0, s], buf[1, s], buf[2, s])
                    wy = (buf[3, s], buf[4, s], buf[5, s])
                    bb = plsc.bitcast(buf[6, s], jnp.int32)
                    bxl = bb >> 10
                    byl = bb & 1023
                    iy = (byl, byl + 1, jnp.minimum(byl + 2, NBY - 1))
                    ix = (bxl, bxl + 1, jnp.minimum(bxl + 2, NBX - 1))
                    for j in range(3):
                        ixl = ix[j] - qlo
                        msk = plsc.bitcast(ixl, jnp.uint32) < jnp.uint32(qrows)
                        for k in range(3):
                            plsc.addupdate_scatter(
                                acc, [ixl, iy[k]], ovx[j] * wy[k], mask=msk)
                return 0

            lax.fori_loop(0, BATCHES // 4, do2, 0)

        for q, (qlo, qrows) in enumerate(((0, 176), (176, 176), (352, 160))):
            start(0, buf0, sem0)

            def zero_row(r, _):
                for cgrp in range(NBY // 16):
                    acc[r, pl.ds(cgrp * 16, 16)] = zeros16
                return 0

            lax.fori_loop(0, qrows, zero_row, 0)

            def ring(c2, _, qlo=qlo, qrows=qrows):
                c = c2 * 2
                wait(c, buf0, sem0)
                start(c + 1, buf1, sem1)
                process(buf0, qlo, qrows)
                wait(c + 1, buf1, sem1)

                @pl.when(c2 < N_CHUNKS // 2 - 1)
                def _():
                    start(c + 2, buf0, sem0)

                process(buf1, qlo, qrows)
                return 0

            lax.fori_loop(0, N_CHUNKS // 2, ring, 0)
            pltpu.sync_copy(acc.at[pl.ds(0, qrows)],
                            out_hbm.at[wid, pl.ds(qlo, qrows)])

    return body


def _tc_reduce(partials):
    def body(p_ref, o_ref):
        o_ref[...] = jnp.sum(p_ref[...], axis=0) * OUT_SCALE

    return pl.pallas_call(
        body,
        grid=(NBX // 8,),
        in_specs=[pl.BlockSpec((N_TILES, 8, NBY), lambda i: (0, i, 0))],
        out_specs=pl.BlockSpec((8, NBY), lambda i: (i, 0)),
        out_shape=jax.ShapeDtypeStruct((NBX, NBY), jnp.float32),
    )(partials)


@jax.jit
def kernel(pos, node_size_x, node_size_y, pin_weights):
    x = pos[:N_PHYS]
    y = pos[N_NODES:N_NODES + N_PHYS]
    nsx = node_size_x[:N_PHYS]
    nsy = node_size_y[:N_PHYS]
    pad = N_PAD - N_PHYS
    half = jnp.full((pad,), 0.5, jnp.float32)
    zero = jnp.zeros((pad,), jnp.float32)
    records = _tc_records(
        jnp.concatenate([x, half]),
        jnp.concatenate([y, half]),
        jnp.concatenate([nsx, half]),
        jnp.concatenate([nsy, half]),
        jnp.concatenate([pin_weights, zero]),
    )
    partials = _sc_partials()(records)
    return _tc_reduce(partials)
